# Initial kernel scaffold; baseline (speedup 1.0000x reference)
import jax
import jax.numpy as jnp
from jax.experimental import pallas as pl


def kernel(x, edge_index, batch, eps1, W1a, b1a, W1b, b1b, g1, be1, eps2, W2a, b2a, W2b, b2b, g2, be2, fcW1, fcb1, fcW2, fcb2):
    raise NotImplementedError("write your pallas kernel here")


# speedup vs baseline: 9.4765x; 9.4765x over previous
import functools

import jax
import jax.numpy as jnp
from jax import lax
from jax.experimental import pallas as pl
from jax.experimental.pallas import tpu as pltpu
from jax.experimental.pallas import tpu_sc as plsc

_NC = 2
_NS = 16
_NW = _NC * _NS
_CH = 80
_G = 64


def _segsum_sc(N, D, E, interpret=False):
    epw = E // _NW
    nchunk = epw // _CH
    assert epw * _NW == E and nchunk * _CH == epw
    wb = 80
    nwb = N // wb
    assert nwb * wb == N and D % 16 == 0

    mesh = plsc.VectorSubcoreMesh(core_axis_name="c", subcore_axis_name="s",
                                  num_cores=_NC, num_subcores=_NS)

    @functools.partial(
        pl.kernel,
        out_type=jax.ShapeDtypeStruct((_NC, N, D), jnp.float32),
        mesh=mesh,
        interpret=interpret,
        compiler_params=pltpu.CompilerParams(use_tc_tiling_on_sc=False),
        scratch_types=[
            pltpu.VMEM((nchunk, _CH), jnp.int32),
            pltpu.VMEM((nchunk, _CH), jnp.int32),
            pltpu.VMEM((_CH, D), jnp.float32),
            pltpu.VMEM((wb, D), jnp.float32),
            pltpu.VMEM_SHARED((N, D), jnp.float32),
            pltpu.SemaphoreType.DMA,
        ],
    )
    def seg(y_hbm, src_hbm, dst_hbm, out_hbm, src_v, dst_v, buf, zbuf, acc, sem):
        c = lax.axis_index("c")
        s = lax.axis_index("s")
        wid = c * _NS + s
        pltpu.sync_copy(src_hbm.at[wid], src_v)
        pltpu.sync_copy(dst_hbm.at[wid], dst_v)

        zvec = jnp.zeros((16,), jnp.float32)

        def zrow(i, carry):
            for k in range(D // 16):
                zbuf[i, pl.ds(k * 16, 16)] = zvec
            return carry

        lax.fori_loop(0, wb, zrow, 0)

        def zchunk(k, carry):
            chunk = s + k * _NS

            @pl.when(chunk < nwb)
            def _():
                pltpu.sync_copy(zbuf, acc.at[pl.ds(chunk * wb, wb)])

            return carry

        lax.fori_loop(0, pl.cdiv(nwb, _NS), zchunk, 0)
        plsc.subcore_barrier()

        def body(j, carry):
            pltpu.async_copy(y_hbm.at[src_v.at[j]], buf, sem).wait()
            pltpu.sync_copy(buf, acc.at[dst_v.at[j]], add=True)
            return carry

        lax.fori_loop(0, nchunk, body, 0)
        plsc.subcore_barrier()

        def wchunk(k, carry):
            chunk = s + k * _NS

            @pl.when(chunk < nwb)
            def _():
                pltpu.sync_copy(acc.at[pl.ds(chunk * wb, wb)],
                                out_hbm.at[c].at[pl.ds(chunk * wb, wb)])

            return carry

        lax.fori_loop(0, pl.cdiv(nwb, _NS), wchunk, 0)

    return seg


def _project(x, W, bn, interpret=False):
    N, K = x.shape
    D = W.shape[1]
    nblk = N // bn

    def body(x_ref, w_ref, o_ref):
        o_ref[...] = jnp.dot(x_ref[...], w_ref[...],
                             preferred_element_type=jnp.float32)

    return pl.pallas_call(
        body,
        grid=(nblk,),
        in_specs=[pl.BlockSpec((bn, K), lambda i: (i, 0)),
                  pl.BlockSpec((K, D), lambda i: (0, 0))],
        out_specs=pl.BlockSpec((bn, D), lambda i: (i, 0)),
        out_shape=jax.ShapeDtypeStruct((N, D), jnp.float32),
        interpret=interpret,
    )(x, W)


def _post_agg(y, p, eps, ba, Wb, bb, bn, interpret=False):
    N, D = y.shape
    nblk = N // bn

    def body(y_ref, p_ref, eps_ref, ba_ref, wb_ref, bb_ref, h_ref, st_ref):
        i = pl.program_id(0)
        e = eps_ref[0, 0]
        pre = (1.0 + e) * y_ref[...] + jnp.sum(p_ref[...], axis=0) + ba_ref[...]
        h = jnp.dot(jax.nn.relu(pre), wb_ref[...],
                    preferred_element_type=jnp.float32) + bb_ref[...]
        h_ref[...] = h
        st = jnp.concatenate([jnp.sum(h, axis=0, keepdims=True),
                              jnp.sum(h * h, axis=0, keepdims=True)], axis=0)

        @pl.when(i == 0)
        def _():
            st_ref[...] = st

        @pl.when(i > 0)
        def _():
            st_ref[...] += st

    return pl.pallas_call(
        body,
        grid=(nblk,),
        in_specs=[pl.BlockSpec((bn, D), lambda i: (i, 0)),
                  pl.BlockSpec((2, bn, D), lambda i: (0, i, 0)),
                  pl.BlockSpec((1, 1), lambda i: (0, 0)),
                  pl.BlockSpec((1, D), lambda i: (0, 0)),
                  pl.BlockSpec((D, D), lambda i: (0, 0)),
                  pl.BlockSpec((1, D), lambda i: (0, 0))],
        out_specs=[pl.BlockSpec((bn, D), lambda i: (i, 0)),
                   pl.BlockSpec((2, D), lambda i: (0, 0))],
        out_shape=[jax.ShapeDtypeStruct((N, D), jnp.float32),
                   jax.ShapeDtypeStruct((2, D), jnp.float32)],
        interpret=interpret,
    )(y, p, eps, ba, Wb, bb)


def _bn_relu_project(h, st, g, be, W, bn, interpret=False):
    N, D = h.shape
    Do = W.shape[1]
    nblk = N // bn

    def body(h_ref, st_ref, g_ref, be_ref, w_ref, o_ref):
        m = st_ref[0:1, :] * (1.0 / N)
        v = st_ref[1:2, :] * (1.0 / N) - m * m
        inv = lax.rsqrt(v + 1e-5) * g_ref[...]
        xn = jax.nn.relu((h_ref[...] - m) * inv + be_ref[...])
        o_ref[...] = jnp.dot(xn, w_ref[...], preferred_element_type=jnp.float32)

    return pl.pallas_call(
        body,
        grid=(nblk,),
        in_specs=[pl.BlockSpec((bn, D), lambda i: (i, 0)),
                  pl.BlockSpec((2, D), lambda i: (0, 0)),
                  pl.BlockSpec((1, D), lambda i: (0, 0)),
                  pl.BlockSpec((1, D), lambda i: (0, 0)),
                  pl.BlockSpec((D, Do), lambda i: (0, 0))],
        out_specs=pl.BlockSpec((bn, Do), lambda i: (i, 0)),
        out_shape=jax.ShapeDtypeStruct((N, Do), jnp.float32),
        interpret=interpret,
    )(h, st, g, be, W)


def _pool_head(h, st, g, be, batchr, fcW1, fcb1, fcW2, fcb2, bn,
               interpret=False):
    N, D = h.shape
    nblk = N // bn

    def body(h_ref, st_ref, g_ref, be_ref, b_ref, w1_ref, b1_ref, w2_ref,
             b2_ref, out_ref, pool_acc, cnt_acc):
        i = pl.program_id(0)
        m = st_ref[0:1, :] * (1.0 / N)
        v = st_ref[1:2, :] * (1.0 / N) - m * m
        inv = lax.rsqrt(v + 1e-5) * g_ref[...]
        x2 = jax.nn.relu((h_ref[...] - m) * inv + be_ref[...])
        oh = (lax.broadcasted_iota(jnp.int32, (_G, bn), 0)
              == b_ref[0]).astype(jnp.float32)
        pool_part = jnp.dot(oh, x2, preferred_element_type=jnp.float32)
        cnt_part = jnp.sum(oh, axis=1, keepdims=True)

        @pl.when(i == 0)
        def _():
            pool_acc[...] = pool_part
            cnt_acc[...] = cnt_part

        @pl.when(i > 0)
        def _():
            pool_acc[...] += pool_part
            cnt_acc[...] += cnt_part

        @pl.when(i == nblk - 1)
        def _():
            pool = pool_acc[...] / jnp.maximum(cnt_acc[...], 1.0)
            hh = jax.nn.relu(jnp.dot(pool, w1_ref[...],
                                     preferred_element_type=jnp.float32)
                             + b1_ref[...]) + pool
            out_ref[...] = jnp.dot(hh, w2_ref[...],
                                   preferred_element_type=jnp.float32) + b2_ref[...]

    return pl.pallas_call(
        body,
        grid=(nblk,),
        in_specs=[pl.BlockSpec((bn, D), lambda i: (i, 0)),
                  pl.BlockSpec((2, D), lambda i: (0, 0)),
                  pl.BlockSpec((1, D), lambda i: (0, 0)),
                  pl.BlockSpec((1, D), lambda i: (0, 0)),
                  pl.BlockSpec((1, 1, bn), lambda i: (i, 0, 0)),
                  pl.BlockSpec(fcW1.shape, lambda i: (0, 0)),
                  pl.BlockSpec((1, fcb1.shape[1]), lambda i: (0, 0)),
                  pl.BlockSpec(fcW2.shape, lambda i: (0, 0)),
                  pl.BlockSpec((1, 1), lambda i: (0, 0))],
        out_specs=pl.BlockSpec((_G, 1), lambda i: (0, 0)),
        out_shape=jax.ShapeDtypeStruct((_G, 1), jnp.float32),
        scratch_shapes=[pltpu.VMEM((_G, D), jnp.float32),
                        pltpu.VMEM((_G, 1), jnp.float32)],
        interpret=interpret,
    )(h, st, g, be, batchr, fcW1, fcb1, fcW2, fcb2)


def _run(x, edge_index, batch, eps1, W1a, b1a, W1b, b1b, g1, be1,
         eps2, W2a, b2a, W2b, b2b, g2, be2, fcW1, fcb1, fcW2, fcb2,
         interpret=False):
    N = x.shape[0]
    E = edge_index.shape[1]
    D1 = W1a.shape[1]
    D2 = W2a.shape[1]
    bn = 1000

    epw = E // _NW
    nchunk = epw // _CH
    src = edge_index[0].reshape(_NW, nchunk, _CH)
    dst = edge_index[1].reshape(_NW, nchunk, _CH)
    batchr = batch.reshape(N // bn, 1, bn)
    r2 = lambda a: a.reshape(1, -1)
    e1 = eps1.reshape(1, 1)
    e2 = eps2.reshape(1, 1)

    y1 = _project(x, W1a, bn, interpret)
    p1 = _segsum_sc(N, D1, E, interpret)(y1, src, dst)
    h1, st1 = _post_agg(y1, p1, e1, r2(b1a), W1b, r2(b1b), bn, interpret)
    y2 = _bn_relu_project(h1, st1, r2(g1), r2(be1), W2a, bn, interpret)
    p2 = _segsum_sc(N, D2, E, interpret)(y2, src, dst)
    h2, st2 = _post_agg(y2, p2, e2, r2(b2a), W2b, r2(b2b), bn, interpret)
    out = _pool_head(h2, st2, r2(g2), r2(be2), batchr, fcW1, r2(fcb1),
                     fcW2, fcb2.reshape(1, 1), bn, interpret)
    return out.reshape(-1)


def kernel(x, edge_index, batch, eps1, W1a, b1a, W1b, b1b, g1, be1,
           eps2, W2a, b2a, W2b, b2b, g2, be2, fcW1, fcb1, fcW2, fcb2):
    return _run(x, edge_index, batch, eps1, W1a, b1a, W1b, b1b, g1, be1,
                eps2, W2a, b2a, W2b, b2b, g2, be2, fcW1, fcb1, fcW2, fcb2)
